# speedup vs baseline: 6.3376x; 6.3376x over previous
import functools

import jax
import jax.numpy as jnp
from jax import lax
from jax.experimental import pallas as pl
from jax.experimental.pallas import tpu as pltpu
from jax.experimental.pallas import tpu_sc as plsc

NUM_ROWS = 4096 * 200
DIM = 128
NC, NS = 2, 16
NW = NC * NS
ROWS_PER_W = NUM_ROWS // NW
CHUNK = 128
STEPS = ROWS_PER_W // CHUNK


def _sc_gather(table, idx):
  mesh = plsc.VectorSubcoreMesh(core_axis_name="c", subcore_axis_name="s")

  @functools.partial(
      pl.kernel,
      out_type=jax.ShapeDtypeStruct((NUM_ROWS, DIM), jnp.float32),
      mesh=mesh,
      scratch_types=[
          pltpu.VMEM((STEPS, CHUNK), jnp.int32),
          pltpu.VMEM((CHUNK, DIM), jnp.float32),
          pltpu.SemaphoreType.DMA,
      ],
  )
  def k(table_hbm, idx_hbm, out_hbm, idx_v, rows_v, sem):
    wid = lax.axis_index("s") * NC + lax.axis_index("c")
    pltpu.sync_copy(idx_hbm.at[wid], idx_v)
    base = wid * ROWS_PER_W

    def body(j, _):
      pltpu.async_copy(table_hbm.at[idx_v.at[j]], rows_v, sem).wait()
      pltpu.sync_copy(rows_v, out_hbm.at[pl.ds(base + j * CHUNK, CHUNK)])
      return 0

    lax.fori_loop(0, STEPS, body, 0)

  return k(table, idx)


def kernel(x, weight):
  idx = x.reshape(NW, STEPS, CHUNK).astype(jnp.int32)
  out = _sc_gather(weight, idx)
  return out.reshape(x.shape + (weight.shape[-1],))


# speedup vs baseline: 9.1531x; 1.4442x over previous
import functools

import jax
import jax.numpy as jnp
from jax import lax
from jax.experimental import pallas as pl
from jax.experimental.pallas import tpu as pltpu
from jax.experimental.pallas import tpu_sc as plsc

NUM_ROWS = 4096 * 200
DIM = 128
NC, NS = 2, 16
NW = NC * NS
ROWS_PER_W = NUM_ROWS // NW
CHUNK = 128
STEPS = ROWS_PER_W // CHUNK
NBUF = 4


def _sc_gather(table, idx):
  mesh = plsc.VectorSubcoreMesh(core_axis_name="c", subcore_axis_name="s")

  @functools.partial(
      pl.kernel,
      out_type=jax.ShapeDtypeStruct((NUM_ROWS, DIM), jnp.float32),
      mesh=mesh,
      scratch_types=[
          pltpu.VMEM((STEPS, CHUNK), jnp.int32),
          *[pltpu.VMEM((CHUNK, DIM), jnp.float32) for _ in range(NBUF)],
          *[pltpu.SemaphoreType.DMA for _ in range(2 * NBUF)],
      ],
  )
  def k(table_hbm, idx_hbm, out_hbm, idx_v, *rest):
    bufs = rest[:NBUF]
    gsems = rest[NBUF:2 * NBUF]
    osems = rest[2 * NBUF:]
    wid = lax.axis_index("s") * NC + lax.axis_index("c")
    pltpu.sync_copy(idx_hbm.at[wid], idx_v)
    base = wid * ROWS_PER_W

    def gwait(b):
      pltpu.make_async_copy(table_hbm.at[idx_v.at[0]], bufs[b], gsems[b]).wait()

    def owait(b):
      pltpu.make_async_copy(bufs[b], out_hbm.at[pl.ds(0, CHUNK)],
                            osems[b]).wait()

    for b in range(NBUF):
      pltpu.async_copy(table_hbm.at[idx_v.at[b]], bufs[b], gsems[b])

    def group(g, _):
      j0 = g * NBUF
      for b in range(NBUF):
        gwait(b)
        pltpu.async_copy(
            bufs[b], out_hbm.at[pl.ds(base + (j0 + b) * CHUNK, CHUNK)],
            osems[b])
      for b in range(NBUF):
        nj = j0 + NBUF + b

        @pl.when(nj < STEPS)
        def _():
          owait(b)
          pltpu.async_copy(table_hbm.at[idx_v.at[nj]], bufs[b], gsems[b])

      return 0

    lax.fori_loop(0, STEPS // NBUF, group, 0)
    for b in range(NBUF):
      owait(b)

  return k(table, idx)


def kernel(x, weight):
  idx = x.reshape(NW, STEPS, CHUNK).astype(jnp.int32)
  out = _sc_gather(weight, idx)
  return out.reshape(x.shape + (weight.shape[-1],))
